# baseline (speedup 1.0000x reference)
import functools

import jax
import jax.numpy as jnp
from jax import lax
from jax.experimental import pallas as pl
from jax.experimental.pallas import tpu as pltpu
from jax.experimental.pallas import tpu_sc as plsc

EMBED = 64
SEQ = 512
EPS = 1e-5
NW = 32
BB = 8


def _make_gather(batch, seq):
    seqs_per_w = batch // NW

    mesh = plsc.VectorSubcoreMesh(core_axis_name="c", subcore_axis_name="s")

    @functools.partial(
        pl.kernel,
        mesh=mesh,
        compiler_params=pltpu.CompilerParams(use_tc_tiling_on_sc=False),
        out_type=jax.ShapeDtypeStruct((batch, seq, EMBED), jnp.float32),
        scratch_types=[
            pltpu.VMEM((seq,), jnp.int32),
            pltpu.VMEM((seq,), jnp.int32),
            pltpu.VMEM((seq, EMBED), jnp.float32),
            pltpu.VMEM((seq, EMBED), jnp.float32),
            pltpu.SemaphoreType.DMA,
            pltpu.SemaphoreType.DMA,
        ],
    )
    def gather(ids_hbm, tok_hbm, out_hbm, idx0, idx1, rows0, rows1,
               sem0, sem1):
        wid = lax.axis_index("s") * 2 + lax.axis_index("c")
        base = wid * seqs_per_w

        def body(i, _):
            s0 = base + i * 2
            s1 = s0 + 1
            pltpu.sync_copy(ids_hbm.at[s0], idx0)
            h0 = pltpu.async_copy(tok_hbm.at[idx0], rows0, sem0)
            pltpu.sync_copy(ids_hbm.at[s1], idx1)
            h1 = pltpu.async_copy(tok_hbm.at[idx1], rows1, sem1)
            h0.wait()
            pltpu.sync_copy(rows0, out_hbm.at[s0])
            h1.wait()
            pltpu.sync_copy(rows1, out_hbm.at[s1])
            return 0

        lax.fori_loop(0, seqs_per_w // 2, body, 0)

    return gather


def _ln_body(x_ref, pos_ref, g_ref, b_ref, o_ref):
    x = x_ref[...] + pos_ref[...][None, :, :]
    mean = jnp.mean(x, axis=-1, keepdims=True)
    var = jnp.mean(x * x, axis=-1, keepdims=True) - mean * mean
    inv = lax.rsqrt(var + EPS)
    o_ref[...] = (x - mean) * inv * g_ref[...] + b_ref[...]


def kernel(input_ids, token_table, pos_table, gamma, beta):
    batch, seq = input_ids.shape

    gathered = _make_gather(batch, seq)(input_ids, token_table)

    out = pl.pallas_call(
        _ln_body,
        grid=(batch // BB,),
        in_specs=[
            pl.BlockSpec((BB, seq, EMBED), lambda i: (i, 0, 0)),
            pl.BlockSpec((seq, EMBED), lambda i: (0, 0)),
            pl.BlockSpec((1, EMBED), lambda i: (0, 0)),
            pl.BlockSpec((1, EMBED), lambda i: (0, 0)),
        ],
        out_specs=pl.BlockSpec((BB, seq, EMBED), lambda i: (i, 0, 0)),
        out_shape=jax.ShapeDtypeStruct((batch, seq, EMBED), jnp.float32),
    )(gathered, pos_table, gamma.reshape(1, EMBED), beta.reshape(1, EMBED))
    return out
